# baseline (speedup 1.0000x reference)
import functools
import math

import jax
import jax.numpy as jnp
from jax import lax
from jax.experimental import pallas as pl
from jax.experimental.pallas import tpu as pltpu
from jax.experimental.pallas import tpu_sc as plsc

D_MODEL = 64
SCALE = math.sqrt(D_MODEL)

NUM_CORES = 2
NUM_SUBCORES = 16
NUM_WORKERS = NUM_CORES * NUM_SUBCORES
LANES = 16

CHUNK = 128
NBUF = 4


def _make_sc_kernel(num_idx_rows: int):
    assert num_idx_rows % NUM_WORKERS == 0
    rows_per_worker = num_idx_rows // NUM_WORKERS
    assert rows_per_worker % NBUF == 0
    groups = rows_per_worker // NBUF
    b_total = num_idx_rows * CHUNK

    mesh = plsc.VectorSubcoreMesh(
        core_axis_name="c", subcore_axis_name="s",
        num_cores=NUM_CORES, num_subcores=NUM_SUBCORES,
    )

    @functools.partial(
        pl.kernel,
        out_type=jax.ShapeDtypeStruct((b_total, D_MODEL), jnp.float32),
        mesh=mesh,
        compiler_params=pltpu.CompilerParams(use_tc_tiling_on_sc=False),
        scratch_types=[
            pltpu.VMEM((rows_per_worker, CHUNK), jnp.int32),
            [pltpu.VMEM((CHUNK, D_MODEL), jnp.float32) for _ in range(NBUF)],
            [pltpu.SemaphoreType.DMA for _ in range(NBUF)],
            [pltpu.SemaphoreType.DMA for _ in range(NBUF)],
        ],
    )
    def sc_kernel(x_hbm, table_hbm, out_hbm, idx_v, bufs, gsems, osems):
        wid = lax.axis_index("s") * NUM_CORES + lax.axis_index("c")
        chunk0 = wid * rows_per_worker

        def gather_desc(c, b):
            return pltpu.make_async_copy(
                table_hbm.at[idx_v.at[c]], bufs[b], gsems[b])

        def out_desc(c, b):
            row0 = (chunk0 + c) * CHUNK
            return pltpu.make_async_copy(
                bufs[b], out_hbm.at[pl.ds(row0, CHUNK)], osems[b])

        def scale_buf(buf):
            @plsc.parallel_loop(0, CHUNK, unroll=2)
            def _(r):
                for c4 in range(D_MODEL // LANES):
                    sl = pl.ds(c4 * LANES, LANES)
                    buf[r, sl] = buf[r, sl] * SCALE

        pltpu.sync_copy(x_hbm.at[pl.ds(chunk0, rows_per_worker)], idx_v)
        for b in range(NBUF):
            gather_desc(b, b).start()

        def group_body(g, _):
            base = g * NBUF
            for b in range(NBUF):
                c = base + b
                gather_desc(c, b).wait()
                scale_buf(bufs[b])
                out_desc(c, b).start()
            for b in range(NBUF):
                c = base + b

                @pl.when(g + 1 < groups)
                def _():
                    out_desc(c, b).wait()
                    gather_desc(c + NBUF, b).start()

            return 0

        lax.fori_loop(0, groups, group_body, 0)
        for b in range(NBUF):
            out_desc((groups - 1) * NBUF + b, b).wait()

    return sc_kernel


def kernel(x, table):
    seq, width = x.shape
    b_total = seq * width
    assert b_total % (NUM_WORKERS * CHUNK) == 0
    num_idx_rows = b_total // CHUNK
    xf = x.reshape(num_idx_rows, CHUNK).astype(jnp.int32)
    out = _make_sc_kernel(num_idx_rows)(xf, table)
    return out.reshape(seq, width, D_MODEL)
